# baseline (speedup 1.0000x reference)
import jax
import jax.numpy as jnp
from jax.experimental import pallas as pl
from jax.experimental.pallas import tpu as pltpu

ALPHA = 0.1
N = 4096
D = 256
BM = 256
NSTEPS = N // BM
NBUF = 6


def _body(adj_hbm, x_hbm, h_hbm, o_hbm, x_v, adj_v, h_v, o_v,
          x_sem, adj_sems, h_sems, o_sems):
    def adj_cp(step, slot):
        return pltpu.make_async_copy(
            adj_hbm.at[pl.ds(step * BM, BM)], adj_v.at[slot], adj_sems.at[slot])

    def h_cp(step, slot):
        return pltpu.make_async_copy(
            h_hbm.at[pl.ds(step * BM, BM)], h_v.at[slot], h_sems.at[slot])

    def o_cp(step, slot):
        return pltpu.make_async_copy(
            o_v.at[slot], o_hbm.at[pl.ds(step * BM, BM)], o_sems.at[slot])

    pltpu.make_async_copy(x_hbm, x_v, x_sem).start()
    for s in range(NBUF):
        adj_cp(s, s).start()
        h_cp(s, s).start()
    pltpu.make_async_copy(x_hbm, x_v, x_sem).wait()

    for step in range(NSTEPS):
        slot = step % NBUF
        adj_cp(step, slot).wait()
        h_cp(step, slot).wait()
        if step >= NBUF:
            o_cp(step - NBUF, slot).wait()
        acc = jnp.dot(adj_v[slot], x_v[...], preferred_element_type=jnp.float32)
        o_v[slot] = (1.0 - ALPHA) * acc + ALPHA * h_v[slot]
        o_cp(step, slot).start()
        nxt = step + NBUF
        if nxt < NSTEPS:
            adj_cp(nxt, slot).start()
            h_cp(nxt, slot).start()

    for step in range(NSTEPS - NBUF, NSTEPS):
        o_cp(step, step % NBUF).wait()


def kernel(x, adj, h):
    out = pl.pallas_call(
        _body,
        in_specs=[
            pl.BlockSpec(memory_space=pltpu.MemorySpace.HBM),
            pl.BlockSpec(memory_space=pltpu.MemorySpace.HBM),
            pl.BlockSpec(memory_space=pltpu.MemorySpace.HBM),
        ],
        out_specs=pl.BlockSpec(memory_space=pltpu.MemorySpace.HBM),
        out_shape=jax.ShapeDtypeStruct((N, D), jnp.float32),
        scratch_shapes=[
            pltpu.VMEM((N, D), jnp.float32),
            pltpu.VMEM((NBUF, BM, N), jnp.float32),
            pltpu.VMEM((NBUF, BM, D), jnp.float32),
            pltpu.VMEM((NBUF, BM, D), jnp.float32),
            pltpu.SemaphoreType.DMA,
            pltpu.SemaphoreType.DMA((NBUF,)),
            pltpu.SemaphoreType.DMA((NBUF,)),
            pltpu.SemaphoreType.DMA((NBUF,)),
        ],
    )(adj, x, h)
    return out


# speedup vs baseline: 1.1031x; 1.1031x over previous
import jax
import jax.numpy as jnp
from jax.experimental import pallas as pl
from jax.experimental.pallas import tpu as pltpu

ALPHA = 0.1
N = 4096
D = 256
BM = 256
NSTEPS = N // BM
NBUF = 3


def _body(adj_hbm, x_hbm, h_hbm, o_hbm, x_v, adj_v, h_v, o_v,
          x_sem, adj_sems, h_sems, o_sems):
    def adj_cp(step, slot):
        return pltpu.make_async_copy(
            adj_hbm.at[pl.ds(step * BM, BM)], adj_v.at[slot], adj_sems.at[slot])

    def h_cp(step, slot):
        return pltpu.make_async_copy(
            h_hbm.at[pl.ds(step * BM, BM)], h_v.at[slot], h_sems.at[slot])

    def o_cp(step, slot):
        return pltpu.make_async_copy(
            o_v.at[slot], o_hbm.at[pl.ds(step * BM, BM)], o_sems.at[slot])

    pltpu.make_async_copy(x_hbm, x_v, x_sem).start()
    for s in range(NBUF):
        adj_cp(s, s).start()
        h_cp(s, s).start()
    pltpu.make_async_copy(x_hbm, x_v, x_sem).wait()

    for step in range(NSTEPS):
        slot = step % NBUF
        adj_cp(step, slot).wait()
        h_cp(step, slot).wait()
        if step >= NBUF:
            o_cp(step - NBUF, slot).wait()
        acc = jnp.dot(adj_v[slot], x_v[...], preferred_element_type=jnp.float32)
        o_v[slot] = (1.0 - ALPHA) * acc + ALPHA * h_v[slot]
        o_cp(step, slot).start()
        nxt = step + NBUF
        if nxt < NSTEPS:
            adj_cp(nxt, slot).start()
            h_cp(nxt, slot).start()

    for step in range(NSTEPS - NBUF, NSTEPS):
        o_cp(step, step % NBUF).wait()


def kernel(x, adj, h):
    out = pl.pallas_call(
        _body,
        in_specs=[
            pl.BlockSpec(memory_space=pltpu.MemorySpace.HBM),
            pl.BlockSpec(memory_space=pltpu.MemorySpace.HBM),
            pl.BlockSpec(memory_space=pltpu.MemorySpace.HBM),
        ],
        out_specs=pl.BlockSpec(memory_space=pltpu.MemorySpace.HBM),
        out_shape=jax.ShapeDtypeStruct((N, D), jnp.float32),
        scratch_shapes=[
            pltpu.VMEM((N, D), jnp.float32),
            pltpu.VMEM((NBUF, BM, N), jnp.float32),
            pltpu.VMEM((NBUF, BM, D), jnp.float32),
            pltpu.VMEM((NBUF, BM, D), jnp.float32),
            pltpu.SemaphoreType.DMA,
            pltpu.SemaphoreType.DMA((NBUF,)),
            pltpu.SemaphoreType.DMA((NBUF,)),
            pltpu.SemaphoreType.DMA((NBUF,)),
        ],
    )(adj, x, h)
    return out
